# Initial kernel scaffold; baseline (speedup 1.0000x reference)
import jax
import jax.numpy as jnp
from jax.experimental import pallas as pl


def kernel(x, edge_index, edge_attr, curr_v_node_id, v_graph_embedding, v_node_dense_embeddings, v_net_batch, params):
    raise NotImplementedError("write your pallas kernel here")


# speedup vs baseline: 2.3933x; 2.3933x over previous
import functools
import jax
import jax.numpy as jnp
import numpy as np
from jax import lax
from jax.experimental import pallas as pl
from jax.experimental.pallas import tpu as pltpu, tpu_sc as plsc

N = 10000
B = 100
NP = 100
E = 320000
D = 128
ED = 4
VN = 10

NT = 32
NPT = 320
NPAD = NT * NPT
CAP = 11264
GB = 64
NBLK = CAP // GB
CH = 2560

_MESH = plsc.VectorSubcoreMesh(core_axis_name="c", subcore_axis_name="s")
_SC_PARAMS = pltpu.CompilerParams(needs_layout_passes=False)


def _leaky(x, s):
    return jnp.where(x >= 0, x, s * x)


def _t_first_body(x_ref, lw_ref, lb_ref, vge_ref, oh_ref, vnd_ref,
                  w_ref, asrc_ref, adst_ref, hw_ref, s_ref, d_ref):
    i = pl.program_id(0)
    h = x_ref[...] @ lw_ref[...] + lb_ref[...]
    h = _leaky(h, 0.01)
    ohb = oh_ref[pl.ds(i * 10, 10), :]
    vndb = vnd_ref[pl.ds(i * 10, 10), :, :]
    cur = jnp.sum(ohb[:, :, None] * vndb, axis=1)
    cond = vge_ref[pl.ds(i * 10, 10), :] + cur
    h = (h.reshape(10, NP, D) + cond[:, None, :]).reshape(10 * NP, D)
    hw = h @ w_ref[...]
    hw_ref[...] = hw
    s_ref[...] = hw @ asrc_ref[...]
    d_ref[...] = hw @ adst_ref[...]


def _tc_first(x, lin_W, lin_b, vge, oh, vnd, W, a_src, a_dst):
    R = 10 * NP
    return pl.pallas_call(
        _t_first_body,
        grid=(10,),
        in_specs=[
            pl.BlockSpec((R, D), lambda i: (i, 0)),
            pl.BlockSpec((D, D), lambda i: (0, 0)),
            pl.BlockSpec((1, D), lambda i: (0, 0)),
            pl.BlockSpec((B, D), lambda i: (0, 0)),
            pl.BlockSpec((B, VN), lambda i: (0, 0)),
            pl.BlockSpec((B, VN, D), lambda i: (0, 0, 0)),
            pl.BlockSpec((D, D), lambda i: (0, 0)),
            pl.BlockSpec((D, 1), lambda i: (0, 0)),
            pl.BlockSpec((D, 1), lambda i: (0, 0)),
        ],
        out_specs=[
            pl.BlockSpec((R, D), lambda i: (i, 0)),
            pl.BlockSpec((R, 1), lambda i: (i, 0)),
            pl.BlockSpec((R, 1), lambda i: (i, 0)),
        ],
        out_shape=[
            jax.ShapeDtypeStruct((N, D), jnp.float32),
            jax.ShapeDtypeStruct((N, 1), jnp.float32),
            jax.ShapeDtypeStruct((N, 1), jnp.float32),
        ],
    )(x, lin_W, lin_b, vge, oh, vnd, W, a_src, a_dst)


def _t_mid_body(msg_ref, att4_ref, wep_ref, bp_ref,
                w_ref, asrc_ref, adst_ref, hw_ref, s_ref, d_ref):
    h = msg_ref[...] + att4_ref[...] @ wep_ref[...] + bp_ref[...]
    h = _leaky(h, 0.01)
    hw = h @ w_ref[...]
    hw_ref[...] = hw
    s_ref[...] = hw @ asrc_ref[...]
    d_ref[...] = hw @ adst_ref[...]


def _tc_mid(msg, att4, We_p, b_p, W, a_src, a_dst):
    R = 1024
    dout = W.shape[1]
    return pl.pallas_call(
        _t_mid_body,
        grid=(10,),
        in_specs=[
            pl.BlockSpec((R, D), lambda i: (i, 0)),
            pl.BlockSpec((R, ED), lambda i: (i, 0)),
            pl.BlockSpec((ED, D), lambda i: (0, 0)),
            pl.BlockSpec((1, D), lambda i: (0, 0)),
            pl.BlockSpec((D, dout), lambda i: (0, 0)),
            pl.BlockSpec((dout, 1), lambda i: (0, 0)),
            pl.BlockSpec((dout, 1), lambda i: (0, 0)),
        ],
        out_specs=[
            pl.BlockSpec((R, dout), lambda i: (i, 0)),
            pl.BlockSpec((R, 1), lambda i: (i, 0)),
            pl.BlockSpec((R, 1), lambda i: (i, 0)),
        ],
        out_shape=[
            jax.ShapeDtypeStruct((NPAD, dout), jnp.float32),
            jax.ShapeDtypeStruct((NPAD, 1), jnp.float32),
            jax.ShapeDtypeStruct((NPAD, 1), jnp.float32),
        ],
    )(msg, att4, We_p, b_p, W, a_src, a_dst)


@functools.partial(
    pl.kernel,
    out_type=(
        jax.ShapeDtypeStruct((NT, CAP), jnp.int32),
        jax.ShapeDtypeStruct((NT, CAP), jnp.int32),
        jax.ShapeDtypeStruct((NT, ED, CAP), jnp.float32),
    ),
    mesh=_MESH,
    compiler_params=_SC_PARAMS,
    scratch_types=[
        pltpu.VMEM((CAP,), jnp.int32),
        pltpu.VMEM((CAP,), jnp.int32),
        pltpu.VMEM((CAP,), jnp.float32),
        pltpu.VMEM((CAP,), jnp.float32),
        pltpu.VMEM((CAP,), jnp.float32),
        pltpu.VMEM((CAP,), jnp.float32),
        pltpu.VMEM((CH,), jnp.int32),
        pltpu.VMEM((CH,), jnp.int32),
        pltpu.VMEM((CH,), jnp.float32),
        pltpu.VMEM((CH,), jnp.float32),
        pltpu.VMEM((CH,), jnp.float32),
        pltpu.VMEM((CH,), jnp.float32),
    ],
)
def _sc_bucket(src_hbm, dst_hbm, eat_hbm, srcc_hbm, dstlc_hbm, eac_hbm,
               src_s, dstl_s, ea0_s, ea1_s, ea2_s, ea3_s,
               srcb, dstb, eb0, eb1, eb2, eb3):
    w = lax.axis_index("s") * 2 + lax.axis_index("c")
    base = w * NPT
    ea_s = (ea0_s, ea1_s, ea2_s, ea3_s)
    eb = (eb0, eb1, eb2, eb3)

    zeros = jnp.zeros((16,), jnp.float32)
    zeros_i = jnp.zeros((16,), jnp.int32)
    dump = jnp.full((16,), NPT, jnp.int32)

    def prefill(i, carry):
        o = i * 16
        src_s[pl.ds(o, 16)] = zeros_i
        dstl_s[pl.ds(o, 16)] = dump
        for j in range(ED):
            ea_s[j][pl.ds(o, 16)] = zeros
        return carry
    lax.fori_loop(0, CAP // 16, prefill, 0)

    def chunk_body(c, cnt):
        pltpu.sync_copy(src_hbm.at[pl.ds(c * CH, CH)], srcb)
        pltpu.sync_copy(dst_hbm.at[pl.ds(c * CH, CH)], dstb)
        for j in range(ED):
            pltpu.sync_copy(eat_hbm.at[j, pl.ds(c * CH, CH)], eb[j])

        def grp(g, cnt):
            o = g * 16
            d16 = dstb[pl.ds(o, 16)]
            msk = (d16 >= base) & (d16 < base + NPT)
            cc = jnp.minimum(cnt, CAP - 16)
            plsc.store_compressed(src_s.at[pl.ds(cc, 16)], srcb[pl.ds(o, 16)],
                                  mask=msk)
            plsc.store_compressed(dstl_s.at[pl.ds(cc, 16)], d16 - base, mask=msk)
            for j in range(ED):
                plsc.store_compressed(ea_s[j].at[pl.ds(cc, 16)],
                                      eb[j][pl.ds(o, 16)], mask=msk)
            pc = plsc.all_reduce_population_count(msk)
            return cnt + pc[0]
        return lax.fori_loop(0, CH // 16, grp, cnt)

    lax.fori_loop(0, E // CH, chunk_body, jnp.int32(0))

    pltpu.sync_copy(src_s, srcc_hbm.at[w])
    pltpu.sync_copy(dstl_s, dstlc_hbm.at[w])
    for j in range(ED):
        pltpu.sync_copy(ea_s[j], eac_hbm.at[w, j])


def _sc_layer_call(sarr, darr, hw, srcc, dstlc, eac, wvec):
    @functools.partial(
        pl.kernel,
        out_type=(
            jax.ShapeDtypeStruct((NPAD, D), jnp.float32),
            jax.ShapeDtypeStruct((NT, (NPT + 8) * 16), jnp.float32),
        ),
        mesh=_MESH,
        compiler_params=_SC_PARAMS,
        scratch_types=[
            pltpu.VMEM((NPAD,), jnp.float32),
            pltpu.VMEM((NPT + 16,), jnp.float32),
            pltpu.VMEM((NBLK, GB), jnp.int32),
            pltpu.VMEM((CAP,), jnp.int32),
            pltpu.VMEM((CAP,), jnp.float32),
            pltpu.VMEM((ED, 1024), jnp.float32),
            pltpu.VMEM((NPT + 8,), jnp.float32),
            pltpu.VMEM(((NPT + 8) * 16,), jnp.float32),
            pltpu.VMEM((NPT + 8, D), jnp.float32),
            pltpu.VMEM((GB, D), jnp.float32),
            pltpu.VMEM((16,), jnp.float32),
            pltpu.SemaphoreType.DMA,
        ],
    )
    def body(s_hbm, d_hbm, hw_hbm, srcc_hbm, dstlc_hbm, eac_hbm, wv_hbm,
             msg_hbm, att4_hbm,
             s_loc, d_own, src_loc, dstl_loc, lg, eab, den, att4_loc,
             out_loc, rows, wvb, sem):
        w = lax.axis_index("s") * 2 + lax.axis_index("c")
        base = w * NPT
        lane = lax.iota(jnp.int32, 16)
        zeros = jnp.zeros((16,), jnp.float32)

        pltpu.sync_copy(s_hbm, s_loc)
        pltpu.sync_copy(d_hbm.at[pl.ds(base, NPT)], d_own.at[pl.ds(0, NPT)])
        d_own[pl.ds(NPT, 16)] = zeros
        pltpu.sync_copy(srcc_hbm.at[w], src_loc)
        pltpu.sync_copy(dstlc_hbm.at[w], dstl_loc)
        pltpu.sync_copy(wv_hbm, wvb)

        wv = wvb[...]
        w0 = lax.reduce_sum(jnp.where(lane == 0, wv, 0.0), (0,))
        w1 = lax.reduce_sum(jnp.where(lane == 1, wv, 0.0), (0,))
        w2 = lax.reduce_sum(jnp.where(lane == 2, wv, 0.0), (0,))
        w3 = lax.reduce_sum(jnp.where(lane == 3, wv, 0.0), (0,))

        def z1(i, c):
            den[pl.ds(i * 16, 16)] = zeros
            return c
        lax.fori_loop(0, (NPT + 8) // 16, z1, 0)

        def z3(r, c):
            for ci in range(D // 16):
                out_loc[r, pl.ds(ci * 16, 16)] = zeros
            att4_loc[pl.ds(r * 16, 16)] = zeros
            return c
        lax.fori_loop(0, NPT + 8, z3, 0)

        def pa_chunk(cc, mx):
            pltpu.sync_copy(eac_hbm.at[w, :, pl.ds(cc * 1024, 1024)], eab)

            def pa_blk(b16, mx):
                bk = cc * 16 + b16

                def pa_grp(go, mx):
                    o = b16 * GB + go * 16
                    fo = bk * GB + go * 16
                    src16 = src_loc[bk, pl.ds(go * 16, 16)]
                    dstl16 = dstl_loc[pl.ds(fo, 16)]
                    sv = plsc.load_gather(s_loc, [src16])
                    dv = plsc.load_gather(d_own, [dstl16])
                    c16 = (eab[0, pl.ds(o, 16)] * w0 + eab[1, pl.ds(o, 16)] * w1
                           + eab[2, pl.ds(o, 16)] * w2 + eab[3, pl.ds(o, 16)] * w3)
                    lgt = sv + dv + c16
                    lgt = jnp.where(lgt >= 0, lgt, 0.2 * lgt)
                    lg[pl.ds(fo, 16)] = lgt
                    return jnp.maximum(mx, lgt)
                return lax.fori_loop(0, GB // 16, pa_grp, mx)
            return lax.fori_loop(0, 16, pa_blk, mx)
        mx = lax.fori_loop(0, CAP // 1024, pa_chunk,
                           jnp.full((16,), -1e30, jnp.float32))
        M = lax.reduce_max(mx, (0,))

        def pb(g, c):
            fo = g * 16
            ex = jnp.exp(lg[pl.ds(fo, 16)] - M)
            lg[pl.ds(fo, 16)] = ex
            dstl16 = dstl_loc[pl.ds(fo, 16)]
            plsc.addupdate_scatter(den, [dstl16], ex)
            return c
        lax.fori_loop(0, CAP // 16, pb, 0)

        def pc_chunk(cc, c):
            pltpu.sync_copy(eac_hbm.at[w, :, pl.ds(cc * 1024, 1024)], eab)

            def pc_grp(gg, c):
                o = gg * 16
                fo = cc * 1024 + o
                ex = lg[pl.ds(fo, 16)]
                dstl16 = dstl_loc[pl.ds(fo, 16)]
                dn = plsc.load_gather(den, [dstl16])
                alpha = ex / (dn + 1e-16)
                lg[pl.ds(fo, 16)] = alpha
                for j in range(ED):
                    plsc.addupdate_scatter(
                        att4_loc, [dstl16 * 16 + j],
                        alpha * eab[j, pl.ds(o, 16)])
                return c
            return lax.fori_loop(0, 64, pc_grp, c)
        lax.fori_loop(0, CAP // 1024, pc_chunk, 0)

        def pd_blk(bk, c):
            pltpu.async_copy(hw_hbm.at[src_loc.at[bk]], rows, sem).wait()

            def pd_grp(g, c):
                fo = bk * GB + g * 16
                alpha = lg[pl.ds(fo, 16)]
                dstl16 = dstl_loc[pl.ds(fo, 16)]
                e16 = g * 16 + lane

                def pd_f(fu, c):
                    for u in range(8):
                        f16 = jnp.full((16,), fu * 8 + u, jnp.int32)
                        v = plsc.load_gather(rows, [e16, f16])
                        plsc.addupdate_scatter(out_loc, [dstl16, f16],
                                               alpha * v)
                    return c
                return lax.fori_loop(0, 16, pd_f, c)
            return lax.fori_loop(0, GB // 16, pd_grp, c)
        lax.fori_loop(0, NBLK, pd_blk, 0)

        pltpu.sync_copy(out_loc.at[pl.ds(0, NPT)], msg_hbm.at[pl.ds(base, NPT)])
        pltpu.sync_copy(att4_loc, att4_hbm.at[w])

    return body(sarr, darr, hw, srcc, dstlc, eac, wvec)


def _sc_last_call(sarr, darr, hw4col, srcc, dstlc, eac, wvec):
    @functools.partial(
        pl.kernel,
        out_type=jax.ShapeDtypeStruct((NPAD,), jnp.float32),
        mesh=_MESH,
        compiler_params=_SC_PARAMS,
        scratch_types=[
            pltpu.VMEM((NPAD,), jnp.float32),
            pltpu.VMEM((NPAD,), jnp.float32),
            pltpu.VMEM((NPT + 16,), jnp.float32),
            pltpu.VMEM((CAP,), jnp.int32),
            pltpu.VMEM((CAP,), jnp.int32),
            pltpu.VMEM((CAP,), jnp.float32),
            pltpu.VMEM((ED, 1024), jnp.float32),
            pltpu.VMEM((NPT + 8,), jnp.float32),
            pltpu.VMEM((NPT + 8,), jnp.float32),
            pltpu.VMEM((16,), jnp.float32),
        ],
    )
    def body(s_hbm, d_hbm, hw_hbm, srcc_hbm, dstlc_hbm, eac_hbm, wv_hbm,
             out_hbm,
             s_loc, hw_loc, d_own, src_loc, dstl_loc, lg, eab, den, out1, wvb):
        w = lax.axis_index("s") * 2 + lax.axis_index("c")
        base = w * NPT
        lane = lax.iota(jnp.int32, 16)
        zeros = jnp.zeros((16,), jnp.float32)

        pltpu.sync_copy(s_hbm, s_loc)
        pltpu.sync_copy(hw_hbm, hw_loc)
        pltpu.sync_copy(d_hbm.at[pl.ds(base, NPT)], d_own.at[pl.ds(0, NPT)])
        d_own[pl.ds(NPT, 16)] = zeros
        pltpu.sync_copy(srcc_hbm.at[w], src_loc)
        pltpu.sync_copy(dstlc_hbm.at[w], dstl_loc)
        pltpu.sync_copy(wv_hbm, wvb)

        wv = wvb[...]
        def lsel(i):
            return lax.reduce_sum(jnp.where(lane == i, wv, 0.0), (0,))
        w0, w1, w2, w3 = lsel(0), lsel(1), lsel(2), lsel(3)
        ae4 = lsel(4)
        b4 = lsel(5)

        def z1(i, c):
            den[pl.ds(i * 16, 16)] = zeros
            out1[pl.ds(i * 16, 16)] = zeros
            return c
        lax.fori_loop(0, (NPT + 8) // 16, z1, 0)

        def pa_chunk(cc, mx):
            pltpu.sync_copy(eac_hbm.at[w, :, pl.ds(cc * 1024, 1024)], eab)

            def pa_grp(gg, mx):
                o = gg * 16
                fo = cc * 1024 + o
                src16 = src_loc[pl.ds(fo, 16)]
                dstl16 = dstl_loc[pl.ds(fo, 16)]
                sv = plsc.load_gather(s_loc, [src16])
                dv = plsc.load_gather(d_own, [dstl16])
                eW = (eab[0, pl.ds(o, 16)] * w0 + eab[1, pl.ds(o, 16)] * w1
                      + eab[2, pl.ds(o, 16)] * w2 + eab[3, pl.ds(o, 16)] * w3)
                lgt = sv + dv + eW * ae4
                lgt = jnp.where(lgt >= 0, lgt, 0.2 * lgt)
                lg[pl.ds(fo, 16)] = lgt
                return jnp.maximum(mx, lgt)
            return lax.fori_loop(0, 64, pa_grp, mx)
        mx = lax.fori_loop(0, CAP // 1024, pa_chunk,
                           jnp.full((16,), -1e30, jnp.float32))
        M = lax.reduce_max(mx, (0,))

        def pb(g, c):
            fo = g * 16
            ex = jnp.exp(lg[pl.ds(fo, 16)] - M)
            lg[pl.ds(fo, 16)] = ex
            plsc.addupdate_scatter(den, [dstl_loc[pl.ds(fo, 16)]], ex)
            return c
        lax.fori_loop(0, CAP // 16, pb, 0)

        def pd_chunk(cc, c):
            pltpu.sync_copy(eac_hbm.at[w, :, pl.ds(cc * 1024, 1024)], eab)

            def pd_grp(gg, c):
                o = gg * 16
                fo = cc * 1024 + o
                ex = lg[pl.ds(fo, 16)]
                src16 = src_loc[pl.ds(fo, 16)]
                dstl16 = dstl_loc[pl.ds(fo, 16)]
                dn = plsc.load_gather(den, [dstl16])
                alpha = ex / (dn + 1e-16)
                eW = (eab[0, pl.ds(o, 16)] * w0 + eab[1, pl.ds(o, 16)] * w1
                      + eab[2, pl.ds(o, 16)] * w2 + eab[3, pl.ds(o, 16)] * w3)
                hsv = plsc.load_gather(hw_loc, [src16])
                plsc.addupdate_scatter(out1, [dstl16], alpha * (hsv + eW))
                return c
            return lax.fori_loop(0, 64, pd_grp, c)
        lax.fori_loop(0, CAP // 1024, pd_chunk, 0)

        def wb(i, c):
            o = i * 16
            out1[pl.ds(o, 16)] = out1[pl.ds(o, 16)] + b4
            return c
        lax.fori_loop(0, NPT // 16, wb, 0)
        pltpu.sync_copy(out1.at[pl.ds(0, NPT)], out_hbm.at[pl.ds(base, NPT)])

    return body(sarr, darr, hw4col, srcc, dstlc, eac, wvec)


def kernel(x, edge_index, edge_attr, curr_v_node_id, v_graph_embedding,
           v_node_dense_embeddings, v_net_batch, params):
    src = edge_index[0]
    dst = edge_index[1]
    eaT = edge_attr.T

    gat = params['gat']
    oh = (curr_v_node_id[:, None] == jnp.arange(VN)[None, :]).astype(jnp.float32)

    srcc, dstlc, eac = _sc_bucket(src, dst, eaT)

    hw, s, d = _tc_first(
        x, params['lin_W'], params['lin_b'].reshape(1, D),
        v_graph_embedding, oh, v_node_dense_embeddings,
        gat[0]['W'], gat[0]['a_src'].reshape(D, 1), gat[0]['a_dst'].reshape(D, 1))

    zpad = jnp.zeros((NPAD - N,), jnp.float32)
    for l in range(4):
        p = gat[l]
        w4 = p['We'] @ p['a_edge']
        wvec = jnp.concatenate([w4, jnp.zeros((12,), jnp.float32)])
        sl = jnp.concatenate([s.reshape(-1)[:N], zpad])
        dl = jnp.concatenate([d.reshape(-1)[:N], zpad])
        msg, att4 = _sc_layer_call(sl, dl, hw, srcc.reshape(NT, NBLK, GB),
                                   dstlc, eac, wvec)
        pn = gat[l + 1]
        dout = pn['W'].shape[1]
        att4 = att4[:, :NPT * 16].reshape(NPAD, 16)[:, :ED]
        hw, s, d = _tc_mid(msg, att4, p['We'], p['b'].reshape(1, D),
                           pn['W'], pn['a_src'].reshape(dout, 1),
                           pn['a_dst'].reshape(dout, 1))

    p4 = gat[4]
    wvec4 = jnp.concatenate([
        p4['We'].reshape(-1), p4['a_edge'].reshape(1), p4['b'].reshape(1),
        jnp.zeros((10,), jnp.float32)])
    out = _sc_last_call(s.reshape(-1), d.reshape(-1), hw.reshape(-1),
                        srcc, dstlc, eac, wvec4)
    return out[:N]


# speedup vs baseline: 4.0622x; 1.6973x over previous
import functools
import jax
import jax.numpy as jnp
import numpy as np
from jax import lax
from jax.experimental import pallas as pl
from jax.experimental.pallas import tpu as pltpu, tpu_sc as plsc

N = 10000
B = 100
NP = 100
E = 320000
D = 128
ED = 4
VN = 10

NT = 32
NPT = 320
NPAD = NT * NPT
CAP = 11264
GB = 64
NBLK = CAP // GB
CH = 2560

_MESH = plsc.VectorSubcoreMesh(core_axis_name="c", subcore_axis_name="s")
_SC_PARAMS = pltpu.CompilerParams(needs_layout_passes=False)


def _leaky(x, s):
    return jnp.where(x >= 0, x, s * x)


def _t_first_body(x_ref, lw_ref, lb_ref, vge_ref, oh_ref, vnd_ref,
                  w_ref, asrc_ref, adst_ref, hw_ref, s_ref, d_ref):
    i = pl.program_id(0)
    h = x_ref[...] @ lw_ref[...] + lb_ref[...]
    h = _leaky(h, 0.01)
    ohb = oh_ref[pl.ds(i * 10, 10), :]
    vndb = vnd_ref[pl.ds(i * 10, 10), :, :]
    cur = jnp.sum(ohb[:, :, None] * vndb, axis=1)
    cond = vge_ref[pl.ds(i * 10, 10), :] + cur
    h = (h.reshape(10, NP, D) + cond[:, None, :]).reshape(10 * NP, D)
    hw = h @ w_ref[...]
    hw_ref[...] = hw
    s_ref[...] = hw @ asrc_ref[...]
    d_ref[...] = hw @ adst_ref[...]


def _tc_first(x, lin_W, lin_b, vge, oh, vnd, W, a_src, a_dst):
    R = 10 * NP
    return pl.pallas_call(
        _t_first_body,
        grid=(10,),
        in_specs=[
            pl.BlockSpec((R, D), lambda i: (i, 0)),
            pl.BlockSpec((D, D), lambda i: (0, 0)),
            pl.BlockSpec((1, D), lambda i: (0, 0)),
            pl.BlockSpec((B, D), lambda i: (0, 0)),
            pl.BlockSpec((B, VN), lambda i: (0, 0)),
            pl.BlockSpec((B, VN, D), lambda i: (0, 0, 0)),
            pl.BlockSpec((D, D), lambda i: (0, 0)),
            pl.BlockSpec((D, 1), lambda i: (0, 0)),
            pl.BlockSpec((D, 1), lambda i: (0, 0)),
        ],
        out_specs=[
            pl.BlockSpec((R, D), lambda i: (i, 0)),
            pl.BlockSpec((R, 1), lambda i: (i, 0)),
            pl.BlockSpec((R, 1), lambda i: (i, 0)),
        ],
        out_shape=[
            jax.ShapeDtypeStruct((N, D), jnp.float32),
            jax.ShapeDtypeStruct((N, 1), jnp.float32),
            jax.ShapeDtypeStruct((N, 1), jnp.float32),
        ],
    )(x, lin_W, lin_b, vge, oh, vnd, W, a_src, a_dst)


def _t_mid_body(msg_ref, att4_ref, wep_ref, bp_ref,
                w_ref, asrc_ref, adst_ref, hw_ref, s_ref, d_ref):
    h = msg_ref[...] + att4_ref[...] @ wep_ref[...] + bp_ref[...]
    h = _leaky(h, 0.01)
    hw = h @ w_ref[...]
    hw_ref[...] = hw
    s_ref[...] = hw @ asrc_ref[...]
    d_ref[...] = hw @ adst_ref[...]


def _tc_mid(msg, att4, We_p, b_p, W, a_src, a_dst):
    R = 1024
    dout = W.shape[1]
    return pl.pallas_call(
        _t_mid_body,
        grid=(10,),
        in_specs=[
            pl.BlockSpec((R, D), lambda i: (i, 0)),
            pl.BlockSpec((R, ED), lambda i: (i, 0)),
            pl.BlockSpec((ED, D), lambda i: (0, 0)),
            pl.BlockSpec((1, D), lambda i: (0, 0)),
            pl.BlockSpec((D, dout), lambda i: (0, 0)),
            pl.BlockSpec((dout, 1), lambda i: (0, 0)),
            pl.BlockSpec((dout, 1), lambda i: (0, 0)),
        ],
        out_specs=[
            pl.BlockSpec((R, dout), lambda i: (i, 0)),
            pl.BlockSpec((R, 1), lambda i: (i, 0)),
            pl.BlockSpec((R, 1), lambda i: (i, 0)),
        ],
        out_shape=[
            jax.ShapeDtypeStruct((NPAD, dout), jnp.float32),
            jax.ShapeDtypeStruct((NPAD, 1), jnp.float32),
            jax.ShapeDtypeStruct((NPAD, 1), jnp.float32),
        ],
    )(msg, att4, We_p, b_p, W, a_src, a_dst)


@functools.partial(
    pl.kernel,
    out_type=(
        jax.ShapeDtypeStruct((NT, CAP), jnp.int32),
        jax.ShapeDtypeStruct((NT, CAP), jnp.int32),
        jax.ShapeDtypeStruct((NT, ED, CAP), jnp.float32),
    ),
    mesh=_MESH,
    compiler_params=_SC_PARAMS,
    scratch_types=[
        pltpu.VMEM((CAP,), jnp.int32),
        pltpu.VMEM((CAP,), jnp.int32),
        pltpu.VMEM((CAP,), jnp.float32),
        pltpu.VMEM((CAP,), jnp.float32),
        pltpu.VMEM((CAP,), jnp.float32),
        pltpu.VMEM((CAP,), jnp.float32),
        pltpu.VMEM((CH,), jnp.int32),
        pltpu.VMEM((CH,), jnp.int32),
        pltpu.VMEM((CH,), jnp.float32),
        pltpu.VMEM((CH,), jnp.float32),
        pltpu.VMEM((CH,), jnp.float32),
        pltpu.VMEM((CH,), jnp.float32),
    ],
)
def _sc_bucket(src_hbm, dst_hbm, eat_hbm, srcc_hbm, dstlc_hbm, eac_hbm,
               src_s, dstl_s, ea0_s, ea1_s, ea2_s, ea3_s,
               srcb, dstb, eb0, eb1, eb2, eb3):
    w = lax.axis_index("s") * 2 + lax.axis_index("c")
    base = w * NPT
    ea_s = (ea0_s, ea1_s, ea2_s, ea3_s)
    eb = (eb0, eb1, eb2, eb3)

    zeros = jnp.zeros((16,), jnp.float32)
    zeros_i = jnp.zeros((16,), jnp.int32)
    dump = jnp.full((16,), NPT, jnp.int32)

    def prefill(i, carry):
        o = i * 16
        src_s[pl.ds(o, 16)] = zeros_i
        dstl_s[pl.ds(o, 16)] = dump
        for j in range(ED):
            ea_s[j][pl.ds(o, 16)] = zeros
        return carry
    lax.fori_loop(0, CAP // 16, prefill, 0)

    def chunk_body(c, cnt):
        pltpu.sync_copy(src_hbm.at[pl.ds(c * CH, CH)], srcb)
        pltpu.sync_copy(dst_hbm.at[pl.ds(c * CH, CH)], dstb)
        for j in range(ED):
            pltpu.sync_copy(eat_hbm.at[j, pl.ds(c * CH, CH)], eb[j])

        def grp(g, cnt):
            o = g * 16
            d16 = dstb[pl.ds(o, 16)]
            msk = (d16 >= base) & (d16 < base + NPT)
            cc = jnp.minimum(cnt, CAP - 16)
            plsc.store_compressed(src_s.at[pl.ds(cc, 16)], srcb[pl.ds(o, 16)],
                                  mask=msk)
            plsc.store_compressed(dstl_s.at[pl.ds(cc, 16)], d16 - base, mask=msk)
            for j in range(ED):
                plsc.store_compressed(ea_s[j].at[pl.ds(cc, 16)],
                                      eb[j][pl.ds(o, 16)], mask=msk)
            pc = plsc.all_reduce_population_count(msk)
            return cnt + pc[0]
        return lax.fori_loop(0, CH // 16, grp, cnt)

    lax.fori_loop(0, E // CH, chunk_body, jnp.int32(0))

    pltpu.sync_copy(src_s, srcc_hbm.at[w])
    pltpu.sync_copy(dstl_s, dstlc_hbm.at[w])
    for j in range(ED):
        pltpu.sync_copy(ea_s[j], eac_hbm.at[w, j])


def _sc_layer_call(sarr, darr, hw, srcc, dstlc, eac, wvec):
    @functools.partial(
        pl.kernel,
        out_type=(
            jax.ShapeDtypeStruct((NPAD, D), jnp.float32),
            jax.ShapeDtypeStruct((NT, ED, NPT + 8), jnp.float32),
        ),
        mesh=_MESH,
        compiler_params=_SC_PARAMS,
        scratch_types=[
            pltpu.VMEM((NPAD,), jnp.float32),
            pltpu.VMEM((NPT + 16,), jnp.float32),
            pltpu.VMEM((NBLK, GB), jnp.int32),
            pltpu.VMEM((CAP,), jnp.int32),
            pltpu.VMEM((CAP,), jnp.float32),
            pltpu.VMEM((ED, 1024), jnp.float32),
            pltpu.VMEM((NPT + 8,), jnp.float32),
            pltpu.VMEM((NPT + 8,), jnp.float32),
            pltpu.VMEM((NPT + 8,), jnp.float32),
            pltpu.VMEM((NPT + 8,), jnp.float32),
            pltpu.VMEM((NPT + 8,), jnp.float32),
            pltpu.VMEM((NPT + 8, D), jnp.float32),
            pltpu.VMEM((GB, D), jnp.float32),
            pltpu.VMEM((16,), jnp.float32),
            pltpu.SemaphoreType.DMA,
        ],
    )
    def body(s_hbm, d_hbm, hw_hbm, srcc_hbm, dstlc_hbm, eac_hbm, wv_hbm,
             msg_hbm, att4_hbm,
             s_loc, d_own, src_loc, dstl_loc, lg, eab, den,
             at40, at41, at42, at43, out_loc, rows, wvb, sem):
        at4 = (at40, at41, at42, at43)
        w = lax.axis_index("s") * 2 + lax.axis_index("c")
        base = w * NPT
        lane = lax.iota(jnp.int32, 16)
        zeros = jnp.zeros((16,), jnp.float32)

        pltpu.sync_copy(s_hbm, s_loc)
        pltpu.sync_copy(d_hbm.at[pl.ds(base, NPT)], d_own.at[pl.ds(0, NPT)])
        d_own[pl.ds(NPT, 16)] = zeros
        pltpu.sync_copy(srcc_hbm.at[w], src_loc)
        pltpu.sync_copy(dstlc_hbm.at[w], dstl_loc)
        pltpu.sync_copy(wv_hbm, wvb)

        wv = wvb[...]
        w0 = lax.reduce_sum(jnp.where(lane == 0, wv, 0.0), (0,))
        w1 = lax.reduce_sum(jnp.where(lane == 1, wv, 0.0), (0,))
        w2 = lax.reduce_sum(jnp.where(lane == 2, wv, 0.0), (0,))
        w3 = lax.reduce_sum(jnp.where(lane == 3, wv, 0.0), (0,))

        def z1(i, c):
            den[pl.ds(i * 16, 16)] = zeros
            for j in range(ED):
                at4[j][pl.ds(i * 16, 16)] = zeros
            return c
        lax.fori_loop(0, (NPT + 8) // 16, z1, 0)

        def z3(r, c):
            for ci in range(D // 16):
                out_loc[r, pl.ds(ci * 16, 16)] = zeros
            return c
        lax.fori_loop(0, NPT + 8, z3, 0)

        def pa_chunk(cc, mx):
            pltpu.sync_copy(eac_hbm.at[w, :, pl.ds(cc * 1024, 1024)], eab)

            def pa_blk(b16, mx):
                bk = cc * 16 + b16

                def pa_grp(go, mx):
                    o = b16 * GB + go * 16
                    fo = bk * GB + go * 16
                    src16 = src_loc[bk, pl.ds(go * 16, 16)]
                    dstl16 = dstl_loc[pl.ds(fo, 16)]
                    sv = plsc.load_gather(s_loc, [src16])
                    dv = plsc.load_gather(d_own, [dstl16])
                    c16 = (eab[0, pl.ds(o, 16)] * w0 + eab[1, pl.ds(o, 16)] * w1
                           + eab[2, pl.ds(o, 16)] * w2 + eab[3, pl.ds(o, 16)] * w3)
                    lgt = sv + dv + c16
                    lgt = jnp.where(lgt >= 0, lgt, 0.2 * lgt)
                    lg[pl.ds(fo, 16)] = lgt
                    return jnp.maximum(mx, lgt)
                return lax.fori_loop(0, GB // 16, pa_grp, mx)
            return lax.fori_loop(0, 16, pa_blk, mx)
        mx = lax.fori_loop(0, CAP // 1024, pa_chunk,
                           jnp.full((16,), -1e30, jnp.float32))
        M = lax.reduce_max(mx, (0,))

        def pb(g, c):
            fo = g * 16
            ex = jnp.exp(lg[pl.ds(fo, 16)] - M)
            lg[pl.ds(fo, 16)] = ex
            dstl16 = dstl_loc[pl.ds(fo, 16)]
            plsc.addupdate_scatter(den, [dstl16], ex)
            return c
        lax.fori_loop(0, CAP // 16, pb, 0)

        def pc_chunk(cc, c):
            pltpu.sync_copy(eac_hbm.at[w, :, pl.ds(cc * 1024, 1024)], eab)

            def pc_grp(gg, c):
                o = gg * 16
                fo = cc * 1024 + o
                ex = lg[pl.ds(fo, 16)]
                dstl16 = dstl_loc[pl.ds(fo, 16)]
                dn = plsc.load_gather(den, [dstl16])
                alpha = ex / (dn + 1e-16)
                lg[pl.ds(fo, 16)] = alpha
                for j in range(ED):
                    plsc.addupdate_scatter(at4[j], [dstl16],
                                           alpha * eab[j, pl.ds(o, 16)])
                return c
            return lax.fori_loop(0, 64, pc_grp, c)
        lax.fori_loop(0, CAP // 1024, pc_chunk, 0)

        def pd_blk(bk, c):
            pltpu.async_copy(hw_hbm.at[src_loc.at[bk]], rows, sem).wait()

            def pd_grp(g, c):
                fo = bk * GB + g * 16
                alpha16 = lg[pl.ds(fo, 16)]
                dstl16 = dstl_loc[pl.ds(fo, 16)]
                for i in range(16):
                    av = jnp.full((16,), alpha16[i], jnp.float32)
                    r = dstl16[i]
                    e = g * 16 + i
                    for ci in range(D // 16):
                        sl = pl.ds(ci * 16, 16)
                        plsc.addupdate(out_loc.at[r, sl], av * rows[e, sl])
                return c
            return lax.fori_loop(0, GB // 16, pd_grp, c)
        lax.fori_loop(0, NBLK, pd_blk, 0)

        pltpu.sync_copy(out_loc.at[pl.ds(0, NPT)], msg_hbm.at[pl.ds(base, NPT)])
        for j in range(ED):
            pltpu.sync_copy(at4[j], att4_hbm.at[w, j])

    return body(sarr, darr, hw, srcc, dstlc, eac, wvec)


def _sc_last_call(sarr, darr, hw4col, srcc, dstlc, eac, wvec):
    @functools.partial(
        pl.kernel,
        out_type=jax.ShapeDtypeStruct((NPAD,), jnp.float32),
        mesh=_MESH,
        compiler_params=_SC_PARAMS,
        scratch_types=[
            pltpu.VMEM((NPAD,), jnp.float32),
            pltpu.VMEM((NPAD,), jnp.float32),
            pltpu.VMEM((NPT + 16,), jnp.float32),
            pltpu.VMEM((CAP,), jnp.int32),
            pltpu.VMEM((CAP,), jnp.int32),
            pltpu.VMEM((CAP,), jnp.float32),
            pltpu.VMEM((ED, 1024), jnp.float32),
            pltpu.VMEM((NPT + 8,), jnp.float32),
            pltpu.VMEM((NPT + 8,), jnp.float32),
            pltpu.VMEM((16,), jnp.float32),
        ],
    )
    def body(s_hbm, d_hbm, hw_hbm, srcc_hbm, dstlc_hbm, eac_hbm, wv_hbm,
             out_hbm,
             s_loc, hw_loc, d_own, src_loc, dstl_loc, lg, eab, den, out1, wvb):
        w = lax.axis_index("s") * 2 + lax.axis_index("c")
        base = w * NPT
        lane = lax.iota(jnp.int32, 16)
        zeros = jnp.zeros((16,), jnp.float32)

        pltpu.sync_copy(s_hbm, s_loc)
        pltpu.sync_copy(hw_hbm, hw_loc)
        pltpu.sync_copy(d_hbm.at[pl.ds(base, NPT)], d_own.at[pl.ds(0, NPT)])
        d_own[pl.ds(NPT, 16)] = zeros
        pltpu.sync_copy(srcc_hbm.at[w], src_loc)
        pltpu.sync_copy(dstlc_hbm.at[w], dstl_loc)
        pltpu.sync_copy(wv_hbm, wvb)

        wv = wvb[...]
        def lsel(i):
            return lax.reduce_sum(jnp.where(lane == i, wv, 0.0), (0,))
        w0, w1, w2, w3 = lsel(0), lsel(1), lsel(2), lsel(3)
        ae4 = lsel(4)
        b4 = lsel(5)

        def z1(i, c):
            den[pl.ds(i * 16, 16)] = zeros
            out1[pl.ds(i * 16, 16)] = zeros
            return c
        lax.fori_loop(0, (NPT + 8) // 16, z1, 0)

        def pa_chunk(cc, mx):
            pltpu.sync_copy(eac_hbm.at[w, :, pl.ds(cc * 1024, 1024)], eab)

            def pa_grp(gg, mx):
                o = gg * 16
                fo = cc * 1024 + o
                src16 = src_loc[pl.ds(fo, 16)]
                dstl16 = dstl_loc[pl.ds(fo, 16)]
                sv = plsc.load_gather(s_loc, [src16])
                dv = plsc.load_gather(d_own, [dstl16])
                eW = (eab[0, pl.ds(o, 16)] * w0 + eab[1, pl.ds(o, 16)] * w1
                      + eab[2, pl.ds(o, 16)] * w2 + eab[3, pl.ds(o, 16)] * w3)
                lgt = sv + dv + eW * ae4
                lgt = jnp.where(lgt >= 0, lgt, 0.2 * lgt)
                lg[pl.ds(fo, 16)] = lgt
                return jnp.maximum(mx, lgt)
            return lax.fori_loop(0, 64, pa_grp, mx)
        mx = lax.fori_loop(0, CAP // 1024, pa_chunk,
                           jnp.full((16,), -1e30, jnp.float32))
        M = lax.reduce_max(mx, (0,))

        def pb(g, c):
            fo = g * 16
            ex = jnp.exp(lg[pl.ds(fo, 16)] - M)
            lg[pl.ds(fo, 16)] = ex
            plsc.addupdate_scatter(den, [dstl_loc[pl.ds(fo, 16)]], ex)
            return c
        lax.fori_loop(0, CAP // 16, pb, 0)

        def pd_chunk(cc, c):
            pltpu.sync_copy(eac_hbm.at[w, :, pl.ds(cc * 1024, 1024)], eab)

            def pd_grp(gg, c):
                o = gg * 16
                fo = cc * 1024 + o
                ex = lg[pl.ds(fo, 16)]
                src16 = src_loc[pl.ds(fo, 16)]
                dstl16 = dstl_loc[pl.ds(fo, 16)]
                dn = plsc.load_gather(den, [dstl16])
                alpha = ex / (dn + 1e-16)
                eW = (eab[0, pl.ds(o, 16)] * w0 + eab[1, pl.ds(o, 16)] * w1
                      + eab[2, pl.ds(o, 16)] * w2 + eab[3, pl.ds(o, 16)] * w3)
                hsv = plsc.load_gather(hw_loc, [src16])
                plsc.addupdate_scatter(out1, [dstl16], alpha * (hsv + eW))
                return c
            return lax.fori_loop(0, 64, pd_grp, c)
        lax.fori_loop(0, CAP // 1024, pd_chunk, 0)

        def wb(i, c):
            o = i * 16
            out1[pl.ds(o, 16)] = out1[pl.ds(o, 16)] + b4
            return c
        lax.fori_loop(0, NPT // 16, wb, 0)
        pltpu.sync_copy(out1.at[pl.ds(0, NPT)], out_hbm.at[pl.ds(base, NPT)])

    return body(sarr, darr, hw4col, srcc, dstlc, eac, wvec)


def kernel(x, edge_index, edge_attr, curr_v_node_id, v_graph_embedding,
           v_node_dense_embeddings, v_net_batch, params):
    src = edge_index[0]
    dst = edge_index[1]
    eaT = edge_attr.T

    gat = params['gat']
    oh = (curr_v_node_id[:, None] == jnp.arange(VN)[None, :]).astype(jnp.float32)

    srcc, dstlc, eac = _sc_bucket(src, dst, eaT)

    hw, s, d = _tc_first(
        x, params['lin_W'], params['lin_b'].reshape(1, D),
        v_graph_embedding, oh, v_node_dense_embeddings,
        gat[0]['W'], gat[0]['a_src'].reshape(D, 1), gat[0]['a_dst'].reshape(D, 1))

    zpad = jnp.zeros((NPAD - N,), jnp.float32)
    for l in range(4):
        p = gat[l]
        w4 = p['We'] @ p['a_edge']
        wvec = jnp.concatenate([w4, jnp.zeros((12,), jnp.float32)])
        sl = jnp.concatenate([s.reshape(-1)[:N], zpad])
        dl = jnp.concatenate([d.reshape(-1)[:N], zpad])
        msg, att4 = _sc_layer_call(sl, dl, hw, srcc.reshape(NT, NBLK, GB),
                                   dstlc, eac, wvec)
        pn = gat[l + 1]
        dout = pn['W'].shape[1]
        att4 = att4[:, :, :NPT].transpose(0, 2, 1).reshape(NPAD, ED)
        hw, s, d = _tc_mid(msg, att4, p['We'], p['b'].reshape(1, D),
                           pn['W'], pn['a_src'].reshape(dout, 1),
                           pn['a_dst'].reshape(dout, 1))

    p4 = gat[4]
    wvec4 = jnp.concatenate([
        p4['We'].reshape(-1), p4['a_edge'].reshape(1), p4['b'].reshape(1),
        jnp.zeros((10,), jnp.float32)])
    out = _sc_last_call(s.reshape(-1), d.reshape(-1), hw.reshape(-1),
                        srcc, dstlc, eac, wvec4)
    return out[:N]


# speedup vs baseline: 4.8052x; 1.1829x over previous
import functools
import jax
import jax.numpy as jnp
import numpy as np
from jax import lax
from jax.experimental import pallas as pl
from jax.experimental.pallas import tpu as pltpu, tpu_sc as plsc

N = 10000
B = 100
NP = 100
E = 320000
D = 128
ED = 4
VN = 10

NT = 32
NPT = 320
NPAD = NT * NPT
CAP = 11264
GB = 64
NBLK = CAP // GB
CH = 2560

_MESH = plsc.VectorSubcoreMesh(core_axis_name="c", subcore_axis_name="s")
_SC_PARAMS = pltpu.CompilerParams(needs_layout_passes=False)


def _leaky(x, s):
    return jnp.where(x >= 0, x, s * x)


def _t_first_body(x_ref, lw_ref, lb_ref, vge_ref, oh_ref, vnd_ref,
                  w_ref, asrc_ref, adst_ref, hw_ref, s_ref, d_ref):
    i = pl.program_id(0)
    h = x_ref[...] @ lw_ref[...] + lb_ref[...]
    h = _leaky(h, 0.01)
    ohb = oh_ref[pl.ds(i * 10, 10), :]
    vndb = vnd_ref[pl.ds(i * 10, 10), :, :]
    cur = jnp.sum(ohb[:, :, None] * vndb, axis=1)
    cond = vge_ref[pl.ds(i * 10, 10), :] + cur
    h = (h.reshape(10, NP, D) + cond[:, None, :]).reshape(10 * NP, D)
    hw = h @ w_ref[...]
    hw_ref[...] = hw
    s_ref[...] = hw @ asrc_ref[...]
    d_ref[...] = hw @ adst_ref[...]


def _tc_first(x, lin_W, lin_b, vge, oh, vnd, W, a_src, a_dst):
    R = 10 * NP
    return pl.pallas_call(
        _t_first_body,
        grid=(10,),
        in_specs=[
            pl.BlockSpec((R, D), lambda i: (i, 0)),
            pl.BlockSpec((D, D), lambda i: (0, 0)),
            pl.BlockSpec((1, D), lambda i: (0, 0)),
            pl.BlockSpec((B, D), lambda i: (0, 0)),
            pl.BlockSpec((B, VN), lambda i: (0, 0)),
            pl.BlockSpec((B, VN, D), lambda i: (0, 0, 0)),
            pl.BlockSpec((D, D), lambda i: (0, 0)),
            pl.BlockSpec((D, 1), lambda i: (0, 0)),
            pl.BlockSpec((D, 1), lambda i: (0, 0)),
        ],
        out_specs=[
            pl.BlockSpec((R, D), lambda i: (i, 0)),
            pl.BlockSpec((R, 1), lambda i: (i, 0)),
            pl.BlockSpec((R, 1), lambda i: (i, 0)),
        ],
        out_shape=[
            jax.ShapeDtypeStruct((N, D), jnp.float32),
            jax.ShapeDtypeStruct((N, 1), jnp.float32),
            jax.ShapeDtypeStruct((N, 1), jnp.float32),
        ],
    )(x, lin_W, lin_b, vge, oh, vnd, W, a_src, a_dst)


def _t_mid_body(msg_ref, att4_ref, wep_ref, bp_ref,
                w_ref, asrc_ref, adst_ref, hw_ref, s_ref, d_ref):
    h = msg_ref[...] + att4_ref[...] @ wep_ref[...] + bp_ref[...]
    h = _leaky(h, 0.01)
    hw = h @ w_ref[...]
    hw_ref[...] = hw
    s_ref[...] = hw @ asrc_ref[...]
    d_ref[...] = hw @ adst_ref[...]


def _tc_mid(msg, att4, We_p, b_p, W, a_src, a_dst):
    R = 1024
    dout = W.shape[1]
    return pl.pallas_call(
        _t_mid_body,
        grid=(10,),
        in_specs=[
            pl.BlockSpec((R, D), lambda i: (i, 0)),
            pl.BlockSpec((R, ED), lambda i: (i, 0)),
            pl.BlockSpec((ED, D), lambda i: (0, 0)),
            pl.BlockSpec((1, D), lambda i: (0, 0)),
            pl.BlockSpec((D, dout), lambda i: (0, 0)),
            pl.BlockSpec((dout, 1), lambda i: (0, 0)),
            pl.BlockSpec((dout, 1), lambda i: (0, 0)),
        ],
        out_specs=[
            pl.BlockSpec((R, dout), lambda i: (i, 0)),
            pl.BlockSpec((R, 1), lambda i: (i, 0)),
            pl.BlockSpec((R, 1), lambda i: (i, 0)),
        ],
        out_shape=[
            jax.ShapeDtypeStruct((NPAD, dout), jnp.float32),
            jax.ShapeDtypeStruct((NPAD, 1), jnp.float32),
            jax.ShapeDtypeStruct((NPAD, 1), jnp.float32),
        ],
    )(msg, att4, We_p, b_p, W, a_src, a_dst)


@functools.partial(
    pl.kernel,
    out_type=(
        jax.ShapeDtypeStruct((NT, CAP), jnp.int32),
        jax.ShapeDtypeStruct((NT, CAP), jnp.int32),
        jax.ShapeDtypeStruct((NT, ED, CAP), jnp.float32),
    ),
    mesh=_MESH,
    compiler_params=_SC_PARAMS,
    scratch_types=[
        pltpu.VMEM((CAP,), jnp.int32),
        pltpu.VMEM((CAP,), jnp.int32),
        pltpu.VMEM((CAP,), jnp.float32),
        pltpu.VMEM((CAP,), jnp.float32),
        pltpu.VMEM((CAP,), jnp.float32),
        pltpu.VMEM((CAP,), jnp.float32),
        pltpu.VMEM((CH,), jnp.int32),
        pltpu.VMEM((CH,), jnp.int32),
        pltpu.VMEM((CH,), jnp.float32),
        pltpu.VMEM((CH,), jnp.float32),
        pltpu.VMEM((CH,), jnp.float32),
        pltpu.VMEM((CH,), jnp.float32),
    ],
)
def _sc_bucket(src_hbm, dst_hbm, eat_hbm, srcc_hbm, dstlc_hbm, eac_hbm,
               src_s, dstl_s, ea0_s, ea1_s, ea2_s, ea3_s,
               srcb, dstb, eb0, eb1, eb2, eb3):
    w = lax.axis_index("s") * 2 + lax.axis_index("c")
    base = w * NPT
    ea_s = (ea0_s, ea1_s, ea2_s, ea3_s)
    eb = (eb0, eb1, eb2, eb3)

    zeros = jnp.zeros((16,), jnp.float32)
    zeros_i = jnp.zeros((16,), jnp.int32)
    dump = jnp.full((16,), NPT, jnp.int32)

    def prefill(i, carry):
        o = i * 16
        src_s[pl.ds(o, 16)] = zeros_i
        dstl_s[pl.ds(o, 16)] = dump
        for j in range(ED):
            ea_s[j][pl.ds(o, 16)] = zeros
        return carry
    lax.fori_loop(0, CAP // 16, prefill, 0)

    def chunk_body(c, cnt):
        pltpu.sync_copy(src_hbm.at[pl.ds(c * CH, CH)], srcb)
        pltpu.sync_copy(dst_hbm.at[pl.ds(c * CH, CH)], dstb)
        for j in range(ED):
            pltpu.sync_copy(eat_hbm.at[j, pl.ds(c * CH, CH)], eb[j])

        def grp(g, cnt):
            o = g * 16
            d16 = dstb[pl.ds(o, 16)]
            msk = (d16 >= base) & (d16 < base + NPT)
            cc = jnp.minimum(cnt, CAP - 16)
            plsc.store_compressed(src_s.at[pl.ds(cc, 16)], srcb[pl.ds(o, 16)],
                                  mask=msk)
            plsc.store_compressed(dstl_s.at[pl.ds(cc, 16)], d16 - base, mask=msk)
            for j in range(ED):
                plsc.store_compressed(ea_s[j].at[pl.ds(cc, 16)],
                                      eb[j][pl.ds(o, 16)], mask=msk)
            pc = plsc.all_reduce_population_count(msk)
            return cnt + pc[0]
        return lax.fori_loop(0, CH // 16, grp, cnt)

    lax.fori_loop(0, E // CH, chunk_body, jnp.int32(0))

    pltpu.sync_copy(src_s, srcc_hbm.at[w])
    pltpu.sync_copy(dstl_s, dstlc_hbm.at[w])
    for j in range(ED):
        pltpu.sync_copy(ea_s[j], eac_hbm.at[w, j])


def _sc_layer_call(sarr, darr, hw, srcc, dstlc, eac, wvec):
    @functools.partial(
        pl.kernel,
        out_type=(
            jax.ShapeDtypeStruct((NPAD, D), jnp.float32),
            jax.ShapeDtypeStruct((NT, ED, NPT + 8), jnp.float32),
        ),
        mesh=_MESH,
        compiler_params=_SC_PARAMS,
        scratch_types=[
            pltpu.VMEM((NPAD,), jnp.float32),
            pltpu.VMEM((NPT + 16,), jnp.float32),
            pltpu.VMEM((NBLK, GB), jnp.int32),
            pltpu.VMEM((CAP,), jnp.int32),
            pltpu.VMEM((CAP,), jnp.float32),
            pltpu.VMEM((ED, 1024), jnp.float32),
            pltpu.VMEM((NPT + 8,), jnp.float32),
            pltpu.VMEM((NPT + 8,), jnp.float32),
            pltpu.VMEM((NPT + 8,), jnp.float32),
            pltpu.VMEM((NPT + 8,), jnp.float32),
            pltpu.VMEM((NPT + 8,), jnp.float32),
            pltpu.VMEM((NPT + 8, D), jnp.float32),
            pltpu.VMEM((GB, D), jnp.float32),
            pltpu.VMEM((GB, D), jnp.float32),
            pltpu.VMEM((16,), jnp.float32),
            pltpu.SemaphoreType.DMA,
            pltpu.SemaphoreType.DMA,
        ],
    )
    def body(s_hbm, d_hbm, hw_hbm, srcc_hbm, dstlc_hbm, eac_hbm, wv_hbm,
             msg_hbm, att4_hbm,
             s_loc, d_own, src_loc, dstl_loc, lg, eab, den,
             at40, at41, at42, at43, out_loc, rows0, rows1, wvb, sem0, sem1):
        at4 = (at40, at41, at42, at43)
        w = lax.axis_index("s") * 2 + lax.axis_index("c")
        base = w * NPT
        lane = lax.iota(jnp.int32, 16)
        zeros = jnp.zeros((16,), jnp.float32)

        pltpu.sync_copy(s_hbm, s_loc)
        pltpu.sync_copy(d_hbm.at[pl.ds(base, NPT)], d_own.at[pl.ds(0, NPT)])
        d_own[pl.ds(NPT, 16)] = zeros
        pltpu.sync_copy(srcc_hbm.at[w], src_loc)
        pltpu.sync_copy(dstlc_hbm.at[w], dstl_loc)
        pltpu.sync_copy(wv_hbm, wvb)

        wv = wvb[...]
        w0 = lax.reduce_sum(jnp.where(lane == 0, wv, 0.0), (0,))
        w1 = lax.reduce_sum(jnp.where(lane == 1, wv, 0.0), (0,))
        w2 = lax.reduce_sum(jnp.where(lane == 2, wv, 0.0), (0,))
        w3 = lax.reduce_sum(jnp.where(lane == 3, wv, 0.0), (0,))

        def z1(i, c):
            den[pl.ds(i * 16, 16)] = zeros
            for j in range(ED):
                at4[j][pl.ds(i * 16, 16)] = zeros
            return c
        lax.fori_loop(0, (NPT + 8) // 16, z1, 0)

        def z3(r, c):
            for ci in range(D // 16):
                out_loc[r, pl.ds(ci * 16, 16)] = zeros
            return c
        lax.fori_loop(0, NPT + 8, z3, 0)

        def pa_chunk(cc, mx):
            pltpu.sync_copy(eac_hbm.at[w, :, pl.ds(cc * 1024, 1024)], eab)

            def pa_grp(g, mx):
                o = g * 16
                fo = cc * 1024 + o
                src16 = src_loc[cc * 16 + (g >> 2), pl.ds((g & 3) * 16, 16)]
                dstl16 = dstl_loc[pl.ds(fo, 16)]
                sv = plsc.load_gather(s_loc, [src16])
                dv = plsc.load_gather(d_own, [dstl16])
                c16 = (eab[0, pl.ds(o, 16)] * w0 + eab[1, pl.ds(o, 16)] * w1
                       + eab[2, pl.ds(o, 16)] * w2 + eab[3, pl.ds(o, 16)] * w3)
                lgt = sv + dv + c16
                lgt = jnp.where(lgt >= 0, lgt, 0.2 * lgt)
                lg[pl.ds(fo, 16)] = lgt
                return jnp.maximum(mx, lgt)
            return plsc.parallel_loop(0, 64, unroll=4, carry=mx)(pa_grp)
        mx = lax.fori_loop(0, CAP // 1024, pa_chunk,
                           jnp.full((16,), -1e30, jnp.float32))
        M = lax.reduce_max(mx, (0,))

        def pb(g):
            fo = g * 16
            ex = jnp.exp(lg[pl.ds(fo, 16)] - M)
            lg[pl.ds(fo, 16)] = ex
            dstl16 = dstl_loc[pl.ds(fo, 16)]
            plsc.addupdate_scatter(den, [dstl16], ex)
        plsc.parallel_loop(0, CAP // 16, unroll=8)(pb)

        def pc_chunk(cc, c):
            pltpu.sync_copy(eac_hbm.at[w, :, pl.ds(cc * 1024, 1024)], eab)

            def pc_grp(gg):
                o = gg * 16
                fo = cc * 1024 + o
                ex = lg[pl.ds(fo, 16)]
                dstl16 = dstl_loc[pl.ds(fo, 16)]
                dn = plsc.load_gather(den, [dstl16])
                alpha = ex / (dn + 1e-16)
                lg[pl.ds(fo, 16)] = alpha
                for j in range(ED):
                    plsc.addupdate_scatter(at4[j], [dstl16],
                                           alpha * eab[j, pl.ds(o, 16)])
            plsc.parallel_loop(0, 64, unroll=4)(pc_grp)
            return c
        lax.fori_loop(0, CAP // 1024, pc_chunk, 0)

        def pd_compute(bk, rows):
            def pd_grp(g):
                fo = bk * GB + g * 16
                alpha16 = lg[pl.ds(fo, 16)]
                dstl16 = dstl_loc[pl.ds(fo, 16)]
                for i in range(16):
                    av = jnp.full((16,), alpha16[i], jnp.float32)
                    r = dstl16[i]
                    e = g * 16 + i
                    for ci in range(D // 16):
                        sl = pl.ds(ci * 16, 16)
                        plsc.addupdate(out_loc.at[r, sl], av * rows[e, sl])
            plsc.parallel_loop(0, GB // 16)(pd_grp)

        pltpu.async_copy(hw_hbm.at[src_loc.at[0]], rows0, sem0)

        def pd_pair(b2, c):
            bk0 = b2 * 2
            bk1 = bk0 + 1
            pltpu.async_copy(hw_hbm.at[src_loc.at[bk1]], rows1, sem1)
            pltpu.make_async_copy(hw_hbm.at[src_loc.at[bk0]], rows0,
                                  sem0).wait()
            pd_compute(bk0, rows0)

            @pl.when(bk1 + 1 < NBLK)
            def _():
                pltpu.async_copy(hw_hbm.at[src_loc.at[bk1 + 1]], rows0, sem0)
            pltpu.make_async_copy(hw_hbm.at[src_loc.at[bk1]], rows1,
                                  sem1).wait()
            pd_compute(bk1, rows1)
            return c
        lax.fori_loop(0, NBLK // 2, pd_pair, 0)

        pltpu.sync_copy(out_loc.at[pl.ds(0, NPT)], msg_hbm.at[pl.ds(base, NPT)])
        for j in range(ED):
            pltpu.sync_copy(at4[j], att4_hbm.at[w, j])

    return body(sarr, darr, hw, srcc, dstlc, eac, wvec)


def _sc_last_call(sarr, darr, hw4col, srcc, dstlc, eac, wvec):
    @functools.partial(
        pl.kernel,
        out_type=jax.ShapeDtypeStruct((NPAD,), jnp.float32),
        mesh=_MESH,
        compiler_params=_SC_PARAMS,
        scratch_types=[
            pltpu.VMEM((NPAD,), jnp.float32),
            pltpu.VMEM((NPAD,), jnp.float32),
            pltpu.VMEM((NPT + 16,), jnp.float32),
            pltpu.VMEM((CAP,), jnp.int32),
            pltpu.VMEM((CAP,), jnp.int32),
            pltpu.VMEM((CAP,), jnp.float32),
            pltpu.VMEM((ED, 1024), jnp.float32),
            pltpu.VMEM((NPT + 8,), jnp.float32),
            pltpu.VMEM((NPT + 8,), jnp.float32),
            pltpu.VMEM((16,), jnp.float32),
        ],
    )
    def body(s_hbm, d_hbm, hw_hbm, srcc_hbm, dstlc_hbm, eac_hbm, wv_hbm,
             out_hbm,
             s_loc, hw_loc, d_own, src_loc, dstl_loc, lg, eab, den, out1, wvb):
        w = lax.axis_index("s") * 2 + lax.axis_index("c")
        base = w * NPT
        lane = lax.iota(jnp.int32, 16)
        zeros = jnp.zeros((16,), jnp.float32)

        pltpu.sync_copy(s_hbm, s_loc)
        pltpu.sync_copy(hw_hbm, hw_loc)
        pltpu.sync_copy(d_hbm.at[pl.ds(base, NPT)], d_own.at[pl.ds(0, NPT)])
        d_own[pl.ds(NPT, 16)] = zeros
        pltpu.sync_copy(srcc_hbm.at[w], src_loc)
        pltpu.sync_copy(dstlc_hbm.at[w], dstl_loc)
        pltpu.sync_copy(wv_hbm, wvb)

        wv = wvb[...]
        def lsel(i):
            return lax.reduce_sum(jnp.where(lane == i, wv, 0.0), (0,))
        w0, w1, w2, w3 = lsel(0), lsel(1), lsel(2), lsel(3)
        ae4 = lsel(4)
        b4 = lsel(5)

        def z1(i, c):
            den[pl.ds(i * 16, 16)] = zeros
            out1[pl.ds(i * 16, 16)] = zeros
            return c
        lax.fori_loop(0, (NPT + 8) // 16, z1, 0)

        def pa_chunk(cc, mx):
            pltpu.sync_copy(eac_hbm.at[w, :, pl.ds(cc * 1024, 1024)], eab)

            def pa_grp(gg, mx):
                o = gg * 16
                fo = cc * 1024 + o
                src16 = src_loc[pl.ds(fo, 16)]
                dstl16 = dstl_loc[pl.ds(fo, 16)]
                sv = plsc.load_gather(s_loc, [src16])
                dv = plsc.load_gather(d_own, [dstl16])
                eW = (eab[0, pl.ds(o, 16)] * w0 + eab[1, pl.ds(o, 16)] * w1
                      + eab[2, pl.ds(o, 16)] * w2 + eab[3, pl.ds(o, 16)] * w3)
                lgt = sv + dv + eW * ae4
                lgt = jnp.where(lgt >= 0, lgt, 0.2 * lgt)
                lg[pl.ds(fo, 16)] = lgt
                return jnp.maximum(mx, lgt)
            return lax.fori_loop(0, 64, pa_grp, mx)
        mx = lax.fori_loop(0, CAP // 1024, pa_chunk,
                           jnp.full((16,), -1e30, jnp.float32))
        M = lax.reduce_max(mx, (0,))

        def pb(g, c):
            fo = g * 16
            ex = jnp.exp(lg[pl.ds(fo, 16)] - M)
            lg[pl.ds(fo, 16)] = ex
            plsc.addupdate_scatter(den, [dstl_loc[pl.ds(fo, 16)]], ex)
            return c
        lax.fori_loop(0, CAP // 16, pb, 0)

        def pd_chunk(cc, c):
            pltpu.sync_copy(eac_hbm.at[w, :, pl.ds(cc * 1024, 1024)], eab)

            def pd_grp(gg, c):
                o = gg * 16
                fo = cc * 1024 + o
                ex = lg[pl.ds(fo, 16)]
                src16 = src_loc[pl.ds(fo, 16)]
                dstl16 = dstl_loc[pl.ds(fo, 16)]
                dn = plsc.load_gather(den, [dstl16])
                alpha = ex / (dn + 1e-16)
                eW = (eab[0, pl.ds(o, 16)] * w0 + eab[1, pl.ds(o, 16)] * w1
                      + eab[2, pl.ds(o, 16)] * w2 + eab[3, pl.ds(o, 16)] * w3)
                hsv = plsc.load_gather(hw_loc, [src16])
                plsc.addupdate_scatter(out1, [dstl16], alpha * (hsv + eW))
                return c
            return lax.fori_loop(0, 64, pd_grp, c)
        lax.fori_loop(0, CAP // 1024, pd_chunk, 0)

        def wb(i, c):
            o = i * 16
            out1[pl.ds(o, 16)] = out1[pl.ds(o, 16)] + b4
            return c
        lax.fori_loop(0, NPT // 16, wb, 0)
        pltpu.sync_copy(out1.at[pl.ds(0, NPT)], out_hbm.at[pl.ds(base, NPT)])

    return body(sarr, darr, hw4col, srcc, dstlc, eac, wvec)


def kernel(x, edge_index, edge_attr, curr_v_node_id, v_graph_embedding,
           v_node_dense_embeddings, v_net_batch, params):
    src = edge_index[0]
    dst = edge_index[1]
    eaT = edge_attr.T

    gat = params['gat']
    oh = (curr_v_node_id[:, None] == jnp.arange(VN)[None, :]).astype(jnp.float32)

    srcc, dstlc, eac = _sc_bucket(src, dst, eaT)

    hw, s, d = _tc_first(
        x, params['lin_W'], params['lin_b'].reshape(1, D),
        v_graph_embedding, oh, v_node_dense_embeddings,
        gat[0]['W'], gat[0]['a_src'].reshape(D, 1), gat[0]['a_dst'].reshape(D, 1))

    zpad = jnp.zeros((NPAD - N,), jnp.float32)
    for l in range(4):
        p = gat[l]
        w4 = p['We'] @ p['a_edge']
        wvec = jnp.concatenate([w4, jnp.zeros((12,), jnp.float32)])
        sl = jnp.concatenate([s.reshape(-1)[:N], zpad])
        dl = jnp.concatenate([d.reshape(-1)[:N], zpad])
        msg, att4 = _sc_layer_call(sl, dl, hw, srcc.reshape(NT, NBLK, GB),
                                   dstlc, eac, wvec)
        pn = gat[l + 1]
        dout = pn['W'].shape[1]
        att4 = att4[:, :, :NPT].transpose(0, 2, 1).reshape(NPAD, ED)
        hw, s, d = _tc_mid(msg, att4, p['We'], p['b'].reshape(1, D),
                           pn['W'], pn['a_src'].reshape(dout, 1),
                           pn['a_dst'].reshape(dout, 1))

    p4 = gat[4]
    wvec4 = jnp.concatenate([
        p4['We'].reshape(-1), p4['a_edge'].reshape(1), p4['b'].reshape(1),
        jnp.zeros((10,), jnp.float32)])
    out = _sc_last_call(s.reshape(-1), d.reshape(-1), hw.reshape(-1),
                        srcc, dstlc, eac, wvec4)
    return out[:N]
